# baseline (speedup 1.0000x reference)
import functools

import jax
import jax.numpy as jnp
from jax.experimental import pallas as pl

_N = 10000
_BM = 400


def _support_body(feat_ref, w1_ref, out_ref):
    out_ref[...] = jnp.dot(feat_ref[...], w1_ref[...],
                           preferred_element_type=jnp.float32,
                           precision=jax.lax.Precision.DEFAULT)


def _pass1_body(adj_ref, s_ref, b1_ref, w2_ref, out_ref):
    acc = jnp.dot(adj_ref[...], s_ref[...],
                  preferred_element_type=jnp.float32,
                  precision=jax.lax.Precision.DEFAULT)
    h = jnp.maximum(acc + b1_ref[...], 0.0)
    out_ref[...] = jnp.dot(h, w2_ref[...],
                           preferred_element_type=jnp.float32,
                           precision=jax.lax.Precision.DEFAULT)


def _pass2_body(adj_ref, hw2_ref, b2_ref, out_ref):
    logits = jnp.dot(adj_ref[...], hw2_ref[...],
                     preferred_element_type=jnp.float32,
                     precision=jax.lax.Precision.DEFAULT) + b2_ref[...]
    m = jnp.max(logits, axis=1, keepdims=True)
    e = jnp.exp(logits - m)
    out_ref[...] = e / jnp.sum(e, axis=1, keepdims=True)


@functools.partial(jax.jit, static_argnames=())
def kernel(feat, adj, W1, b1, W2, b2):
    n, f_in = feat.shape
    h_dim = W1.shape[1]
    c = W2.shape[1]

    support = pl.pallas_call(
        _support_body,
        grid=(n // 2000,),
        in_specs=[
            pl.BlockSpec((2000, f_in), lambda i: (i, 0)),
            pl.BlockSpec((f_in, h_dim), lambda i: (0, 0)),
        ],
        out_specs=pl.BlockSpec((2000, h_dim), lambda i: (i, 0)),
        out_shape=jax.ShapeDtypeStruct((n, h_dim), jnp.float32),
    )(feat, W1)

    hw2 = pl.pallas_call(
        _pass1_body,
        grid=(n // _BM,),
        in_specs=[
            pl.BlockSpec((_BM, n), lambda i: (i, 0)),
            pl.BlockSpec((n, h_dim), lambda i: (0, 0)),
            pl.BlockSpec((1, h_dim), lambda i: (0, 0)),
            pl.BlockSpec((h_dim, c), lambda i: (0, 0)),
        ],
        out_specs=pl.BlockSpec((_BM, c), lambda i: (i, 0)),
        out_shape=jax.ShapeDtypeStruct((n, c), jnp.float32),
    )(adj, support, b1.reshape(1, h_dim), W2)

    out = pl.pallas_call(
        _pass2_body,
        grid=(n // _BM,),
        in_specs=[
            pl.BlockSpec((_BM, n), lambda i: (i, 0)),
            pl.BlockSpec((n, c), lambda i: (0, 0)),
            pl.BlockSpec((1, c), lambda i: (0, 0)),
        ],
        out_specs=pl.BlockSpec((_BM, c), lambda i: (i, 0)),
        out_shape=jax.ShapeDtypeStruct((n, c), jnp.float32),
    )(adj, hw2, b2.reshape(1, c))

    return out


# speedup vs baseline: 1.0585x; 1.0585x over previous
import functools

import jax
import jax.numpy as jnp
from jax.experimental import pallas as pl
from jax.experimental.pallas import tpu as pltpu

_BM = 400


def _body(feat_ref, w1_ref, b1_ref, w2_ref, b2_ref, adj_ref, out_ref,
          s_ref, hw2_ref, *, grid_half):
    i = pl.program_id(0)
    prec = jax.lax.Precision.DEFAULT

    @pl.when(i == 0)
    def _():
        s_ref[...] = jnp.dot(feat_ref[...], w1_ref[...],
                             preferred_element_type=jnp.float32,
                             precision=prec)

    @pl.when(i < grid_half)
    def _():
        acc = jnp.dot(adj_ref[...], s_ref[...],
                      preferred_element_type=jnp.float32, precision=prec)
        h = jnp.maximum(acc + b1_ref[...], 0.0)
        hw2_blk = jnp.dot(h, w2_ref[...],
                          preferred_element_type=jnp.float32, precision=prec)
        hw2_ref[pl.ds(i * _BM, _BM), :] = hw2_blk
        out_ref[...] = hw2_blk

    @pl.when(i >= grid_half)
    def _():
        logits = jnp.dot(adj_ref[...], hw2_ref[...],
                         preferred_element_type=jnp.float32,
                         precision=prec) + b2_ref[...]
        m = jnp.max(logits, axis=1, keepdims=True)
        e = jnp.exp(logits - m)
        out_ref[...] = e / jnp.sum(e, axis=1, keepdims=True)


@jax.jit
def kernel(feat, adj, W1, b1, W2, b2):
    n, f_in = feat.shape
    h_dim = W1.shape[1]
    c = W2.shape[1]
    g = n // _BM

    def adj_idx(i):
        blk = jnp.where(i < g, i, i - g)
        return (blk, 0)

    def out_idx(i):
        blk = jnp.where(i < g, i, i - g)
        return (blk, 0)

    return pl.pallas_call(
        functools.partial(_body, grid_half=g),
        grid=(2 * g,),
        in_specs=[
            pl.BlockSpec((n, f_in), lambda i: (0, 0)),
            pl.BlockSpec((f_in, h_dim), lambda i: (0, 0)),
            pl.BlockSpec((1, h_dim), lambda i: (0, 0)),
            pl.BlockSpec((h_dim, c), lambda i: (0, 0)),
            pl.BlockSpec((1, c), lambda i: (0, 0)),
            pl.BlockSpec((_BM, n), adj_idx),
        ],
        out_specs=pl.BlockSpec((_BM, c), out_idx),
        out_shape=jax.ShapeDtypeStruct((n, c), jnp.float32),
        scratch_shapes=[
            pltpu.VMEM((n, h_dim), jnp.float32),
            pltpu.VMEM((n, c), jnp.float32),
        ],
        compiler_params=pltpu.CompilerParams(
            dimension_semantics=("arbitrary",),
        ),
    )(feat, W1, b1.reshape(1, h_dim), W2, b2.reshape(1, c), adj)
